# baseline (speedup 1.0000x reference)
import functools

import jax
import jax.numpy as jnp
from jax import lax
from jax.experimental import pallas as pl
from jax.experimental.pallas import tpu as pltpu
from jax.experimental.pallas import tpu_sc as plsc

N = 10000
E = 320000
D = 128
H = 256
FC = 512
G = 64

NW = 32
NPW = 320
NPAD = NW * NPW
CH = 8000
NCH = E // CH
ECAP = 335872
SCH = 4096
ACCR = NPW + 8
ACCF = ACCR * H
RB = 1024


def _mesh():
    return plsc.VectorSubcoreMesh(core_axis_name="c", subcore_axis_name="s")


def _worker_id():
    return lax.axis_index("c") * 16 + lax.axis_index("s")


def _sc_partition(src_hbm, dst_hbm, lsrc_hbm, ldst_hbm, cnt_hbm, dinv_hbm,
                  srcbuf, dstbuf, csrc, cdst, deg, dinvbuf, totbuf):
    w = _worker_id()
    lo = w * NPW
    ones_f = jnp.ones((16,), jnp.float32)
    zeros_f = jnp.zeros((16,), jnp.float32)

    def zero_deg(i, carry):
        deg[pl.ds(i * 16, 16)] = zeros_f
        return carry
    lax.fori_loop(0, NPW // 16, zero_deg, 0)

    def chunk_body(k, tot):
        pltpu.sync_copy(src_hbm.at[pl.ds(k * CH, CH)], srcbuf)
        pltpu.sync_copy(dst_hbm.at[pl.ds(k * CH, CH)], dstbuf)

        def grp(i, off):
            sv = srcbuf[pl.ds(i * 16, 16)]
            dv = dstbuf[pl.ds(i * 16, 16)]
            loc = dv - lo
            m = (loc >= 0) & (loc < NPW)
            mi = jnp.where(m, 1, 0).astype(jnp.int32)
            c = plsc.cumsum(mi)
            pos = off + c - 1
            plsc.store_scatter(csrc, [pos], sv, mask=m)
            plsc.store_scatter(cdst, [pos], loc * 256, mask=m)
            plsc.addupdate_scatter(deg, [loc], ones_f, mask=m)
            return off + c[15]

        off = lax.fori_loop(0, CH // 16, grp, jnp.int32(0))
        csrc[pl.ds(off, 16)] = jnp.zeros((16,), jnp.int32)
        cdst[pl.ds(off, 16)] = jnp.zeros((16,), jnp.int32) + NPW * 256
        c_pad = jnp.bitwise_and(off + 15, -16)
        base = pl.multiple_of(w * ECAP + tot, 16)
        pltpu.sync_copy(csrc, lsrc_hbm.at[pl.ds(base, CH + 64)])
        pltpu.sync_copy(cdst, ldst_hbm.at[pl.ds(base, CH + 64)])
        return tot + c_pad

    tot = lax.fori_loop(0, NCH, chunk_body, jnp.int32(0))
    for u in range(8):
        csrc[pl.ds(u * 16, 16)] = jnp.zeros((16,), jnp.int32)
        cdst[pl.ds(u * 16, 16)] = jnp.zeros((16,), jnp.int32) + NPW * 256
    tbase = pl.multiple_of(w * ECAP + tot, 16)
    pltpu.sync_copy(csrc.at[pl.ds(0, 128)], lsrc_hbm.at[pl.ds(tbase, 128)])
    pltpu.sync_copy(cdst.at[pl.ds(0, 128)], ldst_hbm.at[pl.ds(tbase, 128)])
    tot = tot + jnp.bitwise_and(0 - tot, 127)
    totbuf[...] = jnp.zeros((16,), jnp.int32) + tot
    pltpu.sync_copy(totbuf, cnt_hbm.at[pl.ds(w * 16, 16)])

    def dinv_body(i, carry):
        x = deg[pl.ds(i * 16, 16)] + 1.0
        xi = plsc.bitcast(x, jnp.int32)
        y = plsc.bitcast(jnp.int32(0x5F3759DF) - (xi >> 1), jnp.float32)
        y = y * (1.5 - 0.5 * x * y * y)
        y = y * (1.5 - 0.5 * x * y * y)
        y = y * (1.5 - 0.5 * x * y * y)
        y = y * (1.5 - 0.5 * x * y * y)
        dinvbuf[pl.ds(i * 16, 16)] = y
        return carry
    lax.fori_loop(0, NPW // 16, dinv_body, 0)
    pltpu.sync_copy(dinvbuf, dinv_hbm.at[pl.ds(lo, NPW)])


def _make_partition():
    return pl.kernel(
        _sc_partition,
        out_type=(
            jax.ShapeDtypeStruct((NW * ECAP,), jnp.int32),
            jax.ShapeDtypeStruct((NW * ECAP,), jnp.int32),
            jax.ShapeDtypeStruct((NW * 16,), jnp.int32),
            jax.ShapeDtypeStruct((NPAD,), jnp.float32),
        ),
        mesh=_mesh(),
        scratch_types=[
            pltpu.VMEM((CH,), jnp.int32),
            pltpu.VMEM((CH,), jnp.int32),
            pltpu.VMEM((CH + 64,), jnp.int32),
            pltpu.VMEM((CH + 64,), jnp.int32),
            pltpu.VMEM((NPW,), jnp.float32),
            pltpu.VMEM((NPW,), jnp.float32),
            pltpu.VMEM((16,), jnp.int32),
        ],
        compiler_params=pltpu.CompilerParams(needs_layout_passes=False),
    )


BS = 64


def _sc_spmm(g_hbm, lsrc_hbm, ldst_hbm, cnt_hbm, out_hbm,
             srcbuf, dstbuf, rows0, rows1, acc, cntbuf, sem0, sem1):
    w = _worker_id()
    lo = w * NPW
    iota = lax.iota(jnp.int32, 16)
    cols = [iota + cc * 16 for cc in range(16)]
    zeros_f = jnp.zeros((16,), jnp.float32)

    pltpu.sync_copy(cnt_hbm.at[pl.ds(w * 16, 16)], cntbuf)
    tot = cntbuf[...][0]

    def zero_acc(i, carry):
        for u in range(16):
            acc[pl.ds(i * 256 + u * 16, 16)] = zeros_f
        return carry
    lax.fori_loop(0, ACCR, zero_acc, 0)

    def issue(b, rows, sem):
        return pltpu.async_copy(g_hbm.at[srcbuf.at[pl.ds(b * BS, BS)]],
                                rows, sem)

    def wait(rows, sem):
        pltpu.make_async_copy(g_hbm.at[pl.ds(0, BS)], rows, sem).wait()

    def accum(rows, b):
        ebase = b * BS

        def sbody(q, c2):
            dlv = dstbuf[pl.ds(ebase + q * 16, 16)]
            for j in range(16):
                dls = dlv[j]
                r = q * 16 + j
                for cc in range(16):
                    v = rows[r, pl.ds(cc * 16, 16)]
                    plsc.addupdate(acc.at[pl.ds(dls + cc * 16, 16)], v)
            return c2

        lax.fori_loop(0, BS // 16, sbody, 0)

    nk = (tot + (SCH - 1)) >> 12

    def chunk_body(k, carry):
        kbase = pl.multiple_of(w * ECAP + k * SCH, 16)
        pltpu.sync_copy(lsrc_hbm.at[pl.ds(kbase, SCH)], srcbuf)
        pltpu.sync_copy(ldst_hbm.at[pl.ds(kbase, SCH)], dstbuf)
        rem = tot - k * SCH
        nb = jnp.minimum(rem, SCH) >> 6
        npairs = nb >> 1

        issue(0, rows0, sem0)

        def pair(p, c2):
            b0 = 2 * p
            wait(rows0, sem0)
            issue(b0 + 1, rows1, sem1)
            accum(rows0, b0)
            wait(rows1, sem1)
            issue(jnp.minimum(b0 + 2, nb - 1), rows0, sem0)
            accum(rows1, b0 + 1)
            return c2

        lax.fori_loop(0, npairs, pair, 0)
        wait(rows0, sem0)
        return carry

    lax.fori_loop(0, nk, chunk_body, 0)
    pltpu.sync_copy(acc.at[pl.ds(0, NPW * 256)],
                    out_hbm.at[pl.ds(lo * 256, NPW * 256)])


def _make_spmm():
    return pl.kernel(
        _sc_spmm,
        out_type=jax.ShapeDtypeStruct((NPAD * H,), jnp.float32),
        mesh=_mesh(),
        scratch_types=[
            pltpu.VMEM((SCH,), jnp.int32),
            pltpu.VMEM((SCH,), jnp.int32),
            pltpu.VMEM((BS, H), jnp.float32),
            pltpu.VMEM((BS, H), jnp.float32),
            pltpu.VMEM((ACCF,), jnp.float32),
            pltpu.VMEM((16,), jnp.int32),
            pltpu.SemaphoreType.DMA,
            pltpu.SemaphoreType.DMA,
        ],
        compiler_params=pltpu.CompilerParams(needs_layout_passes=False),
    )


def _tc_g0_body(x_ref, w_ref, dinv_ref, o_ref):
    o_ref[...] = (jnp.dot(x_ref[...], w_ref[...],
                          preferred_element_type=jnp.float32) * dinv_ref[...])


def _tc_mid_body(s_ref, g_ref, dinv_ref, b_ref, w_ref, o_ref):
    h = jnp.maximum(dinv_ref[...] * (s_ref[...] + g_ref[...]) + b_ref[...], 0.0)
    o_ref[...] = (jnp.dot(h, w_ref[...],
                          preferred_element_type=jnp.float32) * dinv_ref[...])


def _tc_pool_body(s_ref, g_ref, dinv_ref, b_ref, batch_ref, sums_ref, cnts_ref):
    i = pl.program_id(0)
    h = jnp.maximum(dinv_ref[...] * (s_ref[...] + g_ref[...]) + b_ref[...], 0.0)
    seg = lax.broadcasted_iota(jnp.int32, (1, G), 1)
    onehot = (batch_ref[...] == seg).astype(jnp.float32)
    contrib = lax.dot_general(onehot, h, (((0,), (0,)), ((), ())),
                              preferred_element_type=jnp.float32,
                              precision=lax.Precision.HIGHEST)
    ccount = lax.dot_general(onehot, jnp.ones((RB, H), jnp.float32),
                             (((0,), (0,)), ((), ())),
                             preferred_element_type=jnp.float32,
                             precision=lax.Precision.HIGHEST)

    @pl.when(i == 0)
    def _():
        sums_ref[...] = jnp.zeros_like(sums_ref)
        cnts_ref[...] = jnp.zeros_like(cnts_ref)

    sums_ref[...] += contrib
    cnts_ref[...] += ccount


def _tc_head_body(s_ref, c_ref, w1_ref, b1_ref, w2_ref, b2_ref, o_ref):
    pm = s_ref[...] / jnp.maximum(c_ref[...], 1.0)
    a = jnp.maximum(jnp.dot(pm, w1_ref[...],
                            preferred_element_type=jnp.float32) + b1_ref[...],
                    0.0)
    o_ref[...] = (jnp.dot(a, w2_ref[...],
                          preferred_element_type=jnp.float32) + b2_ref[...])


def _tc_g0(x_p, W0, dinv2d):
    return pl.pallas_call(
        _tc_g0_body,
        grid=(NPAD // RB,),
        in_specs=[
            pl.BlockSpec((RB, D), lambda i: (i, 0)),
            pl.BlockSpec((D, H), lambda i: (0, 0)),
            pl.BlockSpec((RB, 1), lambda i: (i, 0)),
        ],
        out_specs=pl.BlockSpec((RB, H), lambda i: (i, 0)),
        out_shape=jax.ShapeDtypeStruct((NPAD, H), jnp.float32),
    )(x_p, W0, dinv2d)


def _tc_mid(S_prev, g_prev, dinv2d, b_prev, W_next):
    return pl.pallas_call(
        _tc_mid_body,
        grid=(NPAD // RB,),
        in_specs=[
            pl.BlockSpec((RB, H), lambda i: (i, 0)),
            pl.BlockSpec((RB, H), lambda i: (i, 0)),
            pl.BlockSpec((RB, 1), lambda i: (i, 0)),
            pl.BlockSpec((1, H), lambda i: (0, 0)),
            pl.BlockSpec((H, H), lambda i: (0, 0)),
        ],
        out_specs=pl.BlockSpec((RB, H), lambda i: (i, 0)),
        out_shape=jax.ShapeDtypeStruct((NPAD, H), jnp.float32),
    )(S_prev, g_prev, dinv2d, b_prev, W_next)


def _tc_pool(S_prev, g_prev, dinv2d, b_prev, batch_p):
    return pl.pallas_call(
        _tc_pool_body,
        grid=(NPAD // RB,),
        in_specs=[
            pl.BlockSpec((RB, H), lambda i: (i, 0)),
            pl.BlockSpec((RB, H), lambda i: (i, 0)),
            pl.BlockSpec((RB, 1), lambda i: (i, 0)),
            pl.BlockSpec((1, H), lambda i: (0, 0)),
            pl.BlockSpec((RB, 1), lambda i: (i, 0)),
        ],
        out_specs=[
            pl.BlockSpec((G, H), lambda i: (0, 0)),
            pl.BlockSpec((G, H), lambda i: (0, 0)),
        ],
        out_shape=[
            jax.ShapeDtypeStruct((G, H), jnp.float32),
            jax.ShapeDtypeStruct((G, H), jnp.float32),
        ],
    )(S_prev, g_prev, dinv2d, b_prev, batch_p)


def _tc_head(sums, cnts, fc1_W, fc1_b, fc2_W, fc2_b):
    return pl.pallas_call(
        _tc_head_body,
        out_shape=jax.ShapeDtypeStruct((G, 2), jnp.float32),
    )(sums, cnts, fc1_W, fc1_b, fc2_W, fc2_b)


def kernel(x, edge_index, batch, W0, b0, W1, b1, W2, b2,
           fc1_W, fc1_b, fc2_W, fc2_b):
    src = edge_index[0]
    dst = edge_index[1]
    x_p = jnp.pad(x, ((0, NPAD - N), (0, 0)))
    batch_p = jnp.pad(batch, (0, NPAD - N), constant_values=G).reshape(NPAD, 1)

    lsrc, ldst, cnts, dinv = _make_partition()(src, dst)
    dinv2d = dinv.reshape(NPAD, 1)

    spmm = _make_spmm()
    g0 = _tc_g0(x_p, W0, dinv2d)
    S0 = spmm(g0, lsrc, ldst, cnts).reshape(NPAD, H)
    g1 = _tc_mid(S0, g0, dinv2d, b0.reshape(1, H), W1)
    S1 = spmm(g1, lsrc, ldst, cnts).reshape(NPAD, H)
    g2 = _tc_mid(S1, g1, dinv2d, b1.reshape(1, H), W2)
    S2 = spmm(g2, lsrc, ldst, cnts).reshape(NPAD, H)
    sums, cc = _tc_pool(S2, g2, dinv2d, b2.reshape(1, H), batch_p)
    return _tc_head(sums, cc, fc1_W, fc1_b.reshape(1, FC),
                    fc2_W, fc2_b.reshape(1, 2))


# speedup vs baseline: 1.1102x; 1.1102x over previous
import functools

import jax
import jax.numpy as jnp
from jax import lax
from jax.experimental import pallas as pl
from jax.experimental.pallas import tpu as pltpu
from jax.experimental.pallas import tpu_sc as plsc

N = 10000
E = 320000
D = 128
H = 256
FC = 512
G = 64

NW = 32
NPW = 320
NPAD = NW * NPW
CH = 8000
NCH = E // CH
ECAP = 335872
SCH = 4096
ACCR = NPW + 8
ACCF = ACCR * H
RB = 1024


def _mesh():
    return plsc.VectorSubcoreMesh(core_axis_name="c", subcore_axis_name="s")


def _worker_id():
    return lax.axis_index("c") * 16 + lax.axis_index("s")


def _sc_partition(src_hbm, dst_hbm, lsrc_hbm, ldst_hbm, cnt_hbm, dinv_hbm,
                  srcbuf, dstbuf, csrc, cdst, deg, dinvbuf, totbuf):
    w = _worker_id()
    lo = w * NPW
    ones_f = jnp.ones((16,), jnp.float32)
    zeros_f = jnp.zeros((16,), jnp.float32)

    def zero_deg(i, carry):
        deg[pl.ds(i * 16, 16)] = zeros_f
        return carry
    lax.fori_loop(0, NPW // 16, zero_deg, 0)

    def chunk_body(k, tot):
        pltpu.sync_copy(src_hbm.at[pl.ds(k * CH, CH)], srcbuf)
        pltpu.sync_copy(dst_hbm.at[pl.ds(k * CH, CH)], dstbuf)

        def grp(i, off):
            sv = srcbuf[pl.ds(i * 16, 16)]
            dv = dstbuf[pl.ds(i * 16, 16)]
            loc = dv - lo
            m = (loc >= 0) & (loc < NPW)
            mi = jnp.where(m, 1, 0).astype(jnp.int32)
            c = plsc.cumsum(mi)
            pos = off + c - 1
            plsc.store_scatter(csrc, [pos], sv, mask=m)
            plsc.store_scatter(cdst, [pos], loc * 256, mask=m)
            plsc.addupdate_scatter(deg, [loc], ones_f, mask=m)
            return off + c[15]

        off = lax.fori_loop(0, CH // 16, grp, jnp.int32(0))
        csrc[pl.ds(off, 16)] = jnp.zeros((16,), jnp.int32)
        cdst[pl.ds(off, 16)] = jnp.zeros((16,), jnp.int32) + NPW * 256
        c_pad = jnp.bitwise_and(off + 15, -16)
        base = pl.multiple_of(w * ECAP + tot, 16)
        pltpu.sync_copy(csrc, lsrc_hbm.at[pl.ds(base, CH + 64)])
        pltpu.sync_copy(cdst, ldst_hbm.at[pl.ds(base, CH + 64)])
        return tot + c_pad

    tot = lax.fori_loop(0, NCH, chunk_body, jnp.int32(0))
    for u in range(8):
        csrc[pl.ds(u * 16, 16)] = jnp.zeros((16,), jnp.int32)
        cdst[pl.ds(u * 16, 16)] = jnp.zeros((16,), jnp.int32) + NPW * 256
    tbase = pl.multiple_of(w * ECAP + tot, 16)
    pltpu.sync_copy(csrc.at[pl.ds(0, 128)], lsrc_hbm.at[pl.ds(tbase, 128)])
    pltpu.sync_copy(cdst.at[pl.ds(0, 128)], ldst_hbm.at[pl.ds(tbase, 128)])
    tot = tot + jnp.bitwise_and(0 - tot, 127)
    totbuf[...] = jnp.zeros((16,), jnp.int32) + tot
    pltpu.sync_copy(totbuf, cnt_hbm.at[pl.ds(w * 16, 16)])

    def dinv_body(i, carry):
        x = deg[pl.ds(i * 16, 16)] + 1.0
        xi = plsc.bitcast(x, jnp.int32)
        y = plsc.bitcast(jnp.int32(0x5F3759DF) - (xi >> 1), jnp.float32)
        y = y * (1.5 - 0.5 * x * y * y)
        y = y * (1.5 - 0.5 * x * y * y)
        y = y * (1.5 - 0.5 * x * y * y)
        y = y * (1.5 - 0.5 * x * y * y)
        dinvbuf[pl.ds(i * 16, 16)] = y
        return carry
    lax.fori_loop(0, NPW // 16, dinv_body, 0)
    pltpu.sync_copy(dinvbuf, dinv_hbm.at[pl.ds(lo, NPW)])


def _make_partition():
    return pl.kernel(
        _sc_partition,
        out_type=(
            jax.ShapeDtypeStruct((NW * ECAP,), jnp.int32),
            jax.ShapeDtypeStruct((NW * ECAP,), jnp.int32),
            jax.ShapeDtypeStruct((NW * 16,), jnp.int32),
            jax.ShapeDtypeStruct((NPAD,), jnp.float32),
        ),
        mesh=_mesh(),
        scratch_types=[
            pltpu.VMEM((CH,), jnp.int32),
            pltpu.VMEM((CH,), jnp.int32),
            pltpu.VMEM((CH + 64,), jnp.int32),
            pltpu.VMEM((CH + 64,), jnp.int32),
            pltpu.VMEM((NPW,), jnp.float32),
            pltpu.VMEM((NPW,), jnp.float32),
            pltpu.VMEM((16,), jnp.int32),
        ],
        compiler_params=pltpu.CompilerParams(needs_layout_passes=False),
    )


BS = 64


def _sc_spmm(g_hbm, lsrc_hbm, ldst_hbm, cnt_hbm, out_hbm,
             srcbuf, dstbuf, rows0, rows1, acc, cntbuf, sem0, sem1):
    w = _worker_id()
    lo = w * NPW
    iota = lax.iota(jnp.int32, 16)
    cols = [iota + cc * 16 for cc in range(16)]
    zeros_f = jnp.zeros((16,), jnp.float32)

    pltpu.sync_copy(cnt_hbm.at[pl.ds(w * 16, 16)], cntbuf)
    tot = cntbuf[...][0]

    def zero_acc(i, carry):
        for u in range(16):
            acc[pl.ds(i * 256 + u * 16, 16)] = zeros_f
        return carry
    lax.fori_loop(0, ACCR, zero_acc, 0)

    def issue(b, rows, sem):
        return pltpu.async_copy(g_hbm.at[srcbuf.at[pl.ds(b * BS, BS)]],
                                rows, sem)

    def wait(rows, sem):
        pltpu.make_async_copy(g_hbm.at[pl.ds(0, BS)], rows, sem).wait()

    def accum(rows, b):
        ebase = b * BS

        @plsc.parallel_loop(0, BS, unroll=4)
        def ebody(j):
            dl = plsc.load_gather(dstbuf,
                                  [jnp.zeros((16,), jnp.int32) + (ebase + j)])
            for cc in range(16):
                v = rows[j, pl.ds(cc * 16, 16)]
                plsc.addupdate_scatter(acc, [dl + cols[cc]], v)

    nk = (tot + (SCH - 1)) >> 12

    def chunk_body(k, carry):
        kbase = pl.multiple_of(w * ECAP + k * SCH, 16)
        pltpu.sync_copy(lsrc_hbm.at[pl.ds(kbase, SCH)], srcbuf)
        pltpu.sync_copy(ldst_hbm.at[pl.ds(kbase, SCH)], dstbuf)
        rem = tot - k * SCH
        nb = jnp.minimum(rem, SCH) >> 6
        npairs = nb >> 1

        issue(0, rows0, sem0)

        def pair(p, c2):
            b0 = 2 * p
            wait(rows0, sem0)
            issue(b0 + 1, rows1, sem1)
            accum(rows0, b0)
            wait(rows1, sem1)
            issue(jnp.minimum(b0 + 2, nb - 1), rows0, sem0)
            accum(rows1, b0 + 1)
            return c2

        lax.fori_loop(0, npairs, pair, 0)
        wait(rows0, sem0)
        return carry

    lax.fori_loop(0, nk, chunk_body, 0)
    pltpu.sync_copy(acc.at[pl.ds(0, NPW * 256)],
                    out_hbm.at[pl.ds(lo * 256, NPW * 256)])


def _make_spmm():
    return pl.kernel(
        _sc_spmm,
        out_type=jax.ShapeDtypeStruct((NPAD * H,), jnp.float32),
        mesh=_mesh(),
        scratch_types=[
            pltpu.VMEM((SCH,), jnp.int32),
            pltpu.VMEM((SCH,), jnp.int32),
            pltpu.VMEM((BS, H), jnp.float32),
            pltpu.VMEM((BS, H), jnp.float32),
            pltpu.VMEM((ACCF,), jnp.float32),
            pltpu.VMEM((16,), jnp.int32),
            pltpu.SemaphoreType.DMA,
            pltpu.SemaphoreType.DMA,
        ],
        compiler_params=pltpu.CompilerParams(needs_layout_passes=False),
    )


def _tc_g0_body(x_ref, w_ref, dinv_ref, o_ref):
    o_ref[...] = (jnp.dot(x_ref[...], w_ref[...],
                          preferred_element_type=jnp.float32) * dinv_ref[...])


def _tc_mid_body(s_ref, g_ref, dinv_ref, b_ref, w_ref, o_ref):
    h = jnp.maximum(dinv_ref[...] * (s_ref[...] + g_ref[...]) + b_ref[...], 0.0)
    o_ref[...] = (jnp.dot(h, w_ref[...],
                          preferred_element_type=jnp.float32) * dinv_ref[...])


def _tc_pool_body(s_ref, g_ref, dinv_ref, b_ref, batch_ref, sums_ref, cnts_ref):
    i = pl.program_id(0)
    h = jnp.maximum(dinv_ref[...] * (s_ref[...] + g_ref[...]) + b_ref[...], 0.0)
    seg = lax.broadcasted_iota(jnp.int32, (1, G), 1)
    onehot = (batch_ref[...] == seg).astype(jnp.float32)
    contrib = lax.dot_general(onehot, h, (((0,), (0,)), ((), ())),
                              preferred_element_type=jnp.float32,
                              precision=lax.Precision.HIGHEST)
    ccount = lax.dot_general(onehot, jnp.ones((RB, H), jnp.float32),
                             (((0,), (0,)), ((), ())),
                             preferred_element_type=jnp.float32,
                             precision=lax.Precision.HIGHEST)

    @pl.when(i == 0)
    def _():
        sums_ref[...] = jnp.zeros_like(sums_ref)
        cnts_ref[...] = jnp.zeros_like(cnts_ref)

    sums_ref[...] += contrib
    cnts_ref[...] += ccount


def _tc_head_body(s_ref, c_ref, w1_ref, b1_ref, w2_ref, b2_ref, o_ref):
    pm = s_ref[...] / jnp.maximum(c_ref[...], 1.0)
    a = jnp.maximum(jnp.dot(pm, w1_ref[...],
                            preferred_element_type=jnp.float32) + b1_ref[...],
                    0.0)
    o_ref[...] = (jnp.dot(a, w2_ref[...],
                          preferred_element_type=jnp.float32) + b2_ref[...])


def _tc_g0(x_p, W0, dinv2d):
    return pl.pallas_call(
        _tc_g0_body,
        grid=(NPAD // RB,),
        in_specs=[
            pl.BlockSpec((RB, D), lambda i: (i, 0)),
            pl.BlockSpec((D, H), lambda i: (0, 0)),
            pl.BlockSpec((RB, 1), lambda i: (i, 0)),
        ],
        out_specs=pl.BlockSpec((RB, H), lambda i: (i, 0)),
        out_shape=jax.ShapeDtypeStruct((NPAD, H), jnp.float32),
    )(x_p, W0, dinv2d)


def _tc_mid(S_prev, g_prev, dinv2d, b_prev, W_next):
    return pl.pallas_call(
        _tc_mid_body,
        grid=(NPAD // RB,),
        in_specs=[
            pl.BlockSpec((RB, H), lambda i: (i, 0)),
            pl.BlockSpec((RB, H), lambda i: (i, 0)),
            pl.BlockSpec((RB, 1), lambda i: (i, 0)),
            pl.BlockSpec((1, H), lambda i: (0, 0)),
            pl.BlockSpec((H, H), lambda i: (0, 0)),
        ],
        out_specs=pl.BlockSpec((RB, H), lambda i: (i, 0)),
        out_shape=jax.ShapeDtypeStruct((NPAD, H), jnp.float32),
    )(S_prev, g_prev, dinv2d, b_prev, W_next)


def _tc_pool(S_prev, g_prev, dinv2d, b_prev, batch_p):
    return pl.pallas_call(
        _tc_pool_body,
        grid=(NPAD // RB,),
        in_specs=[
            pl.BlockSpec((RB, H), lambda i: (i, 0)),
            pl.BlockSpec((RB, H), lambda i: (i, 0)),
            pl.BlockSpec((RB, 1), lambda i: (i, 0)),
            pl.BlockSpec((1, H), lambda i: (0, 0)),
            pl.BlockSpec((RB, 1), lambda i: (i, 0)),
        ],
        out_specs=[
            pl.BlockSpec((G, H), lambda i: (0, 0)),
            pl.BlockSpec((G, H), lambda i: (0, 0)),
        ],
        out_shape=[
            jax.ShapeDtypeStruct((G, H), jnp.float32),
            jax.ShapeDtypeStruct((G, H), jnp.float32),
        ],
    )(S_prev, g_prev, dinv2d, b_prev, batch_p)


def _tc_head(sums, cnts, fc1_W, fc1_b, fc2_W, fc2_b):
    return pl.pallas_call(
        _tc_head_body,
        out_shape=jax.ShapeDtypeStruct((G, 2), jnp.float32),
    )(sums, cnts, fc1_W, fc1_b, fc2_W, fc2_b)


def kernel(x, edge_index, batch, W0, b0, W1, b1, W2, b2,
           fc1_W, fc1_b, fc2_W, fc2_b):
    src = edge_index[0]
    dst = edge_index[1]
    x_p = jnp.pad(x, ((0, NPAD - N), (0, 0)))
    batch_p = jnp.pad(batch, (0, NPAD - N), constant_values=G).reshape(NPAD, 1)

    lsrc, ldst, cnts, dinv = _make_partition()(src, dst)
    dinv2d = dinv.reshape(NPAD, 1)

    spmm = _make_spmm()
    g0 = _tc_g0(x_p, W0, dinv2d)
    S0 = spmm(g0, lsrc, ldst, cnts).reshape(NPAD, H)
    g1 = _tc_mid(S0, g0, dinv2d, b0.reshape(1, H), W1)
    S1 = spmm(g1, lsrc, ldst, cnts).reshape(NPAD, H)
    g2 = _tc_mid(S1, g1, dinv2d, b1.reshape(1, H), W2)
    S2 = spmm(g2, lsrc, ldst, cnts).reshape(NPAD, H)
    sums, cc = _tc_pool(S2, g2, dinv2d, b2.reshape(1, H), batch_p)
    return _tc_head(sums, cc, fc1_W, fc1_b.reshape(1, FC),
                    fc2_W, fc2_b.reshape(1, 2))
